# baseline (speedup 1.0000x reference)
import functools

import jax
import jax.numpy as jnp
from jax import lax
from jax.experimental import pallas as pl
from jax.experimental.pallas import tpu as pltpu
from jax.experimental.pallas import tpu_sc as plsc


@functools.lru_cache(maxsize=None)
def _build_warp(B, C, H, W):
    HW = H * W
    N = B * HW
    CH = C // 2
    info = plsc.get_sparse_core_info()
    NC = info.num_cores
    NW = NC * info.num_subcores
    L = info.num_lanes
    K = 128
    assert N % NW == 0 and C % (2 * L) == 0
    PPW = N // NW
    assert PPW % K == 0
    NCHUNK = PPW // K
    assert NCHUNK % 2 == 0
    GPC = K // L
    LOG_HW = HW.bit_length() - 1
    assert (1 << LOG_HW) == HW and (1 << (W.bit_length() - 1)) == W

    mesh = plsc.VectorSubcoreMesh(core_axis_name="core", subcore_axis_name="sub")

    @functools.partial(
        pl.kernel,
        out_type=jax.ShapeDtypeStruct((B, C, HW), jnp.float32),
        mesh=mesh,
        compiler_params=pltpu.CompilerParams(
            use_tc_tiling_on_sc=False, needs_layout_passes=False),
        scratch_types=[
            pltpu.VMEM((PPW,), jnp.float32),
            pltpu.VMEM((PPW,), jnp.float32),
            pltpu.VMEM((4, K), jnp.int32),
            pltpu.VMEM((4, K), jnp.int32),
            pltpu.VMEM((4, K), jnp.float32),
            pltpu.VMEM((4, K), jnp.float32),
            pltpu.VMEM((4, K, CH), jnp.int32),
            pltpu.VMEM((4, K, CH), jnp.int32),
            pltpu.VMEM((C, K), jnp.float32),
            pltpu.VMEM((C, K), jnp.float32),
            pltpu.SemaphoreType.DMA,
            pltpu.SemaphoreType.DMA,
        ],
    )
    def warp(table, fx_hbm, fy_hbm, out_hbm,
             fxv, fyv, idxA, idxB, wgtA, wgtB, rA, rB, outA, outB,
             sem_g, sem_w):
        wid = lax.axis_index("sub") * NC + lax.axis_index("core")
        base = wid * PPW
        lanes = lax.iota(jnp.int32, L)

        pltpu.sync_copy(fx_hbm.at[pl.ds(base, PPW)], fxv)
        pltpu.sync_copy(fy_hbm.at[pl.ds(base, PPW)], fyv)

        def phase1(ci, idxr, wgtr):
            p0 = base + ci * K
            f0 = ci * K
            for g in range(GPC):
                s = g * L
                sl = pl.ds(s, L)
                p = p0 + s + lanes
                x = p & (W - 1)
                y = (p >> (W.bit_length() - 1)) & (H - 1)
                boff = p - (p & (HW - 1))
                xf = x.astype(jnp.float32) + fxv[pl.ds(f0 + s, L)]
                yf = y.astype(jnp.float32) + fyv[pl.ds(f0 + s, L)]
                xfc = jnp.clip(xf, -1.0, float(W))
                yfc = jnp.clip(yf, -1.0, float(H))
                xt = xfc.astype(jnp.int32)
                yt = yfc.astype(jnp.int32)
                x0i = jnp.where(xt.astype(jnp.float32) > xfc, xt - 1, xt)
                y0i = jnp.where(yt.astype(jnp.float32) > yfc, yt - 1, yt)
                a = xfc - x0i.astype(jnp.float32)
                b = yfc - y0i.astype(jnp.float32)
                x0 = jnp.clip(x0i, 0, W - 1)
                x1 = jnp.clip(x0i + 1, 0, W - 1)
                y0 = jnp.clip(y0i, 0, H - 1)
                y1 = jnp.clip(y0i + 1, 0, H - 1)
                r0 = boff + y0 * W
                r1 = boff + y1 * W
                idxr[0, sl] = r0 + x0
                idxr[1, sl] = r0 + x1
                idxr[2, sl] = r1 + x0
                idxr[3, sl] = r1 + x1
                ia = 1.0 - a
                ib = 1.0 - b
                wgtr[0, sl] = ia * ib
                wgtr[1, sl] = a * ib
                wgtr[2, sl] = ia * b
                wgtr[3, sl] = a * b

        def issue_gathers(idxr, rr):
            for j in range(4):
                pltpu.async_copy(table.at[idxr.at[j]], rr.at[j], sem_g)

        def drain_gathers(idxr, rr):
            for j in range(4):
                pltpu.make_async_copy(table.at[idxr.at[j]], rr.at[j], sem_g).wait()

        def blend(rr, wgtr, outr):
            hi_mask = jnp.full((L,), -65536, jnp.int32)
            UNROLL = 4
            for g in range(GPC):
                sl16 = pl.ds(g * L, L)
                pix = lanes + g * L
                wv0 = wgtr[0, sl16]
                wv1 = wgtr[1, sl16]
                wv2 = wgtr[2, sl16]
                wv3 = wgtr[3, sl16]
                jidx = [jnp.full((L,), j, jnp.int32) for j in range(4)]

                def wbody(wq, c2):
                    w0 = wq * UNROLL
                    for u in range(UNROLL):
                        w = w0 + u
                        ws = lax.broadcast(w, (L,))
                        v = [plsc.load_gather(rr, [jidx[j], pix, ws])
                             for j in range(4)]
                        lo = [plsc.bitcast(vv << 16, jnp.float32) for vv in v]
                        hi = [plsc.bitcast(vv & hi_mask, jnp.float32)
                              for vv in v]
                        lov = (wv0 * lo[0] + wv1 * lo[1]
                               + wv2 * lo[2] + wv3 * lo[3])
                        hiv = (wv0 * hi[0] + wv1 * hi[1]
                               + wv2 * hi[2] + wv3 * hi[3])
                        outr[2 * w, sl16] = lov
                        outr[2 * w + 1, sl16] = hiv
                    return c2

                lax.fori_loop(0, CH // UNROLL, wbody, 0)

        def out_slice(ci):
            p0 = base + ci * K
            bb = p0 >> LOG_HW
            yx0 = pl.multiple_of(p0 - (bb << LOG_HW), K)
            return out_hbm.at[bb, :, pl.ds(yx0, K)]

        phase1(0, idxA, wgtA)
        issue_gathers(idxA, rA)
        half = NCHUNK // 2

        def step(cp, carry):
            c0 = 2 * cp
            phase1(c0 + 1, idxB, wgtB)
            issue_gathers(idxB, rB)
            drain_gathers(idxA, rA)

            @pl.when(cp >= 1)
            def _():
                pltpu.make_async_copy(outA, out_slice(c0 - 2), sem_w).wait()

            blend(rA, wgtA, outA)
            pltpu.async_copy(outA, out_slice(c0), sem_w)

            @pl.when(cp < half - 1)
            def _():
                phase1(c0 + 2, idxA, wgtA)
                issue_gathers(idxA, rA)

            drain_gathers(idxB, rB)

            @pl.when(cp >= 1)
            def _():
                pltpu.make_async_copy(outB, out_slice(c0 - 1), sem_w).wait()

            blend(rB, wgtB, outB)
            pltpu.async_copy(outB, out_slice(c0 + 1), sem_w)
            return carry

        lax.fori_loop(0, half, step, 0)
        pltpu.make_async_copy(outA, out_slice(NCHUNK - 2), sem_w).wait()
        pltpu.make_async_copy(outB, out_slice(NCHUNK - 1), sem_w).wait()

    return warp


def kernel(input1, input2):
    B, C, H, W = input1.shape
    CH = C // 2
    t = input1.transpose(0, 2, 3, 1).astype(jnp.bfloat16)
    table = lax.bitcast_convert_type(
        t.reshape(B * H * W, CH, 2), jnp.int32)
    fx = input2[:, 0].reshape(-1)
    fy = input2[:, 1].reshape(-1)
    out = _build_warp(B, C, H, W)(table, fx, fy)
    return out.reshape(B, C, H, W)


# speedup vs baseline: 1.8905x; 1.8905x over previous
import functools

import jax
import jax.numpy as jnp
from jax import lax
from jax.experimental import pallas as pl
from jax.experimental.pallas import tpu as pltpu
from jax.experimental.pallas import tpu_sc as plsc


@functools.lru_cache(maxsize=None)
def _build_warp(B, C, H, W):
    HW = H * W
    N = B * HW
    CH = C // 2
    info = plsc.get_sparse_core_info()
    NC = info.num_cores
    NW = NC * info.num_subcores
    L = info.num_lanes
    K = 128
    assert N % NW == 0 and C % (2 * L) == 0
    PPW = N // NW
    assert PPW % K == 0
    NCHUNK = PPW // K
    assert NCHUNK % 2 == 0
    GPC = K // L
    LOG_HW = HW.bit_length() - 1
    assert (1 << LOG_HW) == HW and (1 << (W.bit_length() - 1)) == W

    mesh = plsc.VectorSubcoreMesh(core_axis_name="core", subcore_axis_name="sub")

    @functools.partial(
        pl.kernel,
        out_type=jax.ShapeDtypeStruct((B, C, HW), jnp.float32),
        mesh=mesh,
        compiler_params=pltpu.CompilerParams(
            use_tc_tiling_on_sc=False, needs_layout_passes=False),
        scratch_types=[
            pltpu.VMEM((PPW,), jnp.float32),
            pltpu.VMEM((PPW,), jnp.float32),
            pltpu.VMEM((4, K), jnp.int32),
            pltpu.VMEM((4, K), jnp.int32),
            pltpu.VMEM((4, K), jnp.float32),
            pltpu.VMEM((4, K), jnp.float32),
            pltpu.VMEM((4, K, CH), jnp.int32),
            pltpu.VMEM((4, K, CH), jnp.int32),
            pltpu.VMEM((C, K), jnp.float32),
            pltpu.VMEM((C, K), jnp.float32),
            pltpu.SemaphoreType.DMA,
            pltpu.SemaphoreType.DMA,
        ],
    )
    def warp(table, flow_hbm, out_hbm,
             fxv, fyv, idxA, idxB, wgtA, wgtB, rA, rB, outA, outB,
             sem_g, sem_w):
        wid = lax.axis_index("sub") * NC + lax.axis_index("core")
        base = wid * PPW
        lanes = lax.iota(jnp.int32, L)

        wb = base >> LOG_HW
        wyx = pl.multiple_of(base - (wb << LOG_HW), K)
        pltpu.sync_copy(flow_hbm.at[2 * wb, pl.ds(wyx, PPW)], fxv)
        pltpu.sync_copy(flow_hbm.at[2 * wb + 1, pl.ds(wyx, PPW)], fyv)

        def phase1(ci, idxr, wgtr):
            p0 = base + ci * K
            f0 = ci * K
            for g in range(GPC):
                s = g * L
                sl = pl.ds(s, L)
                p = p0 + s + lanes
                x = p & (W - 1)
                y = (p >> (W.bit_length() - 1)) & (H - 1)
                boff = p - (p & (HW - 1))
                xf = x.astype(jnp.float32) + fxv[pl.ds(f0 + s, L)]
                yf = y.astype(jnp.float32) + fyv[pl.ds(f0 + s, L)]
                xfc = jnp.clip(xf, -1.0, float(W))
                yfc = jnp.clip(yf, -1.0, float(H))
                xt = xfc.astype(jnp.int32)
                yt = yfc.astype(jnp.int32)
                x0i = jnp.where(xt.astype(jnp.float32) > xfc, xt - 1, xt)
                y0i = jnp.where(yt.astype(jnp.float32) > yfc, yt - 1, yt)
                a = xfc - x0i.astype(jnp.float32)
                b = yfc - y0i.astype(jnp.float32)
                x0 = jnp.clip(x0i, 0, W - 1)
                x1 = jnp.clip(x0i + 1, 0, W - 1)
                y0 = jnp.clip(y0i, 0, H - 1)
                y1 = jnp.clip(y0i + 1, 0, H - 1)
                r0 = boff + y0 * W
                r1 = boff + y1 * W
                idxr[0, sl] = r0 + x0
                idxr[1, sl] = r0 + x1
                idxr[2, sl] = r1 + x0
                idxr[3, sl] = r1 + x1
                ia = 1.0 - a
                ib = 1.0 - b
                wgtr[0, sl] = ia * ib
                wgtr[1, sl] = a * ib
                wgtr[2, sl] = ia * b
                wgtr[3, sl] = a * b

        def issue_gathers(idxr, rr):
            for j in range(4):
                pltpu.async_copy(table.at[idxr.at[j]], rr.at[j], sem_g)

        def drain_gathers(idxr, rr):
            for j in range(4):
                pltpu.make_async_copy(table.at[idxr.at[j]], rr.at[j], sem_g).wait()

        def blend(rr, wgtr, outr):
            hi_mask = jnp.full((L,), -65536, jnp.int32)
            UNROLL = 4
            for g in range(GPC):
                sl16 = pl.ds(g * L, L)
                pix = lanes + g * L
                wv0 = wgtr[0, sl16]
                wv1 = wgtr[1, sl16]
                wv2 = wgtr[2, sl16]
                wv3 = wgtr[3, sl16]
                jidx = [jnp.full((L,), j, jnp.int32) for j in range(4)]

                @plsc.parallel_loop(0, CH, unroll=UNROLL)
                def _(w):
                    ws = lax.broadcast(w, (L,))
                    v = [plsc.load_gather(rr, [jidx[j], pix, ws])
                         for j in range(4)]
                    lo = [plsc.bitcast(vv << 16, jnp.float32) for vv in v]
                    hi = [plsc.bitcast(vv & hi_mask, jnp.float32)
                          for vv in v]
                    lov = (wv0 * lo[0] + wv1 * lo[1]
                           + wv2 * lo[2] + wv3 * lo[3])
                    hiv = (wv0 * hi[0] + wv1 * hi[1]
                           + wv2 * hi[2] + wv3 * hi[3])
                    outr[2 * w, sl16] = lov
                    outr[2 * w + 1, sl16] = hiv

        def out_slice(ci):
            p0 = base + ci * K
            bb = p0 >> LOG_HW
            yx0 = pl.multiple_of(p0 - (bb << LOG_HW), K)
            return out_hbm.at[bb, :, pl.ds(yx0, K)]

        phase1(0, idxA, wgtA)
        issue_gathers(idxA, rA)
        half = NCHUNK // 2

        def step(cp, carry):
            c0 = 2 * cp
            phase1(c0 + 1, idxB, wgtB)
            issue_gathers(idxB, rB)
            drain_gathers(idxA, rA)

            @pl.when(cp >= 1)
            def _():
                pltpu.make_async_copy(outA, out_slice(c0 - 2), sem_w).wait()

            blend(rA, wgtA, outA)
            pltpu.async_copy(outA, out_slice(c0), sem_w)

            @pl.when(cp < half - 1)
            def _():
                phase1(c0 + 2, idxA, wgtA)
                issue_gathers(idxA, rA)

            drain_gathers(idxB, rB)

            @pl.when(cp >= 1)
            def _():
                pltpu.make_async_copy(outB, out_slice(c0 - 1), sem_w).wait()

            blend(rB, wgtB, outB)
            pltpu.async_copy(outB, out_slice(c0 + 1), sem_w)
            return carry

        lax.fori_loop(0, half, step, 0)
        pltpu.make_async_copy(outA, out_slice(NCHUNK - 2), sem_w).wait()
        pltpu.make_async_copy(outB, out_slice(NCHUNK - 1), sem_w).wait()

    return warp


def kernel(input1, input2):
    B, C, H, W = input1.shape
    u = lax.bitcast_convert_type(input1, jnp.uint32)
    rne = (u + jnp.uint32(0x7FFF) + ((u >> 16) & 1)) >> 16
    words = (rne[:, 0::2] | (rne[:, 1::2] << 16)).astype(jnp.int32)
    table = words.transpose(0, 2, 3, 1).reshape(B * H * W, C // 2)
    flow = input2.reshape(B * 2, H * W)
    out = _build_warp(B, C, H, W)(table, flow)
    return out.reshape(B, C, H, W)


# speedup vs baseline: 2.0610x; 1.0902x over previous
import functools

import jax
import jax.numpy as jnp
from jax import lax
from jax.experimental import pallas as pl
from jax.experimental.pallas import tpu as pltpu
from jax.experimental.pallas import tpu_sc as plsc


@functools.lru_cache(maxsize=None)
def _build_warp(B, C, H, W):
    HW = H * W
    N = B * HW
    CH = C // 2
    info = plsc.get_sparse_core_info()
    NC = info.num_cores
    NW = NC * info.num_subcores
    L = info.num_lanes
    K = 128
    assert N % NW == 0 and C % (2 * L) == 0
    PPW = N // NW
    assert PPW % K == 0
    NCHUNK = PPW // K
    assert NCHUNK % 2 == 0
    GPC = K // L
    LOG_HW = HW.bit_length() - 1
    assert (1 << LOG_HW) == HW and (1 << (W.bit_length() - 1)) == W

    mesh = plsc.VectorSubcoreMesh(core_axis_name="core", subcore_axis_name="sub")

    @functools.partial(
        pl.kernel,
        out_type=jax.ShapeDtypeStruct((B, C, HW), jnp.float32),
        mesh=mesh,
        compiler_params=pltpu.CompilerParams(
            use_tc_tiling_on_sc=False, needs_layout_passes=False),
        scratch_types=[
            pltpu.VMEM((PPW,), jnp.float32),
            pltpu.VMEM((PPW,), jnp.float32),
            pltpu.VMEM((4, K), jnp.int32),
            pltpu.VMEM((4, K), jnp.int32),
            pltpu.VMEM((4, K), jnp.float32),
            pltpu.VMEM((4, K), jnp.float32),
            pltpu.VMEM((4, K, CH), jnp.int32),
            pltpu.VMEM((4, K, CH), jnp.int32),
            pltpu.VMEM((C, K), jnp.float32),
            pltpu.VMEM((C, K), jnp.float32),
            pltpu.SemaphoreType.DMA,
            pltpu.SemaphoreType.DMA,
        ],
    )
    def warp(table, flow_hbm, out_hbm,
             fxv, fyv, idxA, idxB, wgtA, wgtB, rA, rB, outA, outB,
             sem_g, sem_w):
        wid = lax.axis_index("sub") * NC + lax.axis_index("core")
        base = wid * PPW
        lanes = lax.iota(jnp.int32, L)

        wb = base >> LOG_HW
        wyx = pl.multiple_of(base - (wb << LOG_HW), K)
        pltpu.sync_copy(flow_hbm.at[2 * wb, pl.ds(wyx, PPW)], fxv)
        pltpu.sync_copy(flow_hbm.at[2 * wb + 1, pl.ds(wyx, PPW)], fyv)

        def phase1(ci, idxr, wgtr):
            p0 = base + ci * K
            f0 = ci * K
            for g in range(GPC):
                s = g * L
                sl = pl.ds(s, L)
                p = p0 + s + lanes
                x = p & (W - 1)
                y = (p >> (W.bit_length() - 1)) & (H - 1)
                boff = p - (p & (HW - 1))
                xf = x.astype(jnp.float32) + fxv[pl.ds(f0 + s, L)]
                yf = y.astype(jnp.float32) + fyv[pl.ds(f0 + s, L)]
                xfc = jnp.clip(xf, -1.0, float(W))
                yfc = jnp.clip(yf, -1.0, float(H))
                xt = xfc.astype(jnp.int32)
                yt = yfc.astype(jnp.int32)
                x0i = jnp.where(xt.astype(jnp.float32) > xfc, xt - 1, xt)
                y0i = jnp.where(yt.astype(jnp.float32) > yfc, yt - 1, yt)
                a = xfc - x0i.astype(jnp.float32)
                b = yfc - y0i.astype(jnp.float32)
                x0 = jnp.clip(x0i, 0, W - 1)
                x1 = jnp.clip(x0i + 1, 0, W - 1)
                y0 = jnp.clip(y0i, 0, H - 1)
                y1 = jnp.clip(y0i + 1, 0, H - 1)
                r0 = boff + y0 * W
                r1 = boff + y1 * W
                idxr[0, sl] = r0 + x0
                idxr[1, sl] = r0 + x1
                idxr[2, sl] = r1 + x0
                idxr[3, sl] = r1 + x1
                ia = 1.0 - a
                ib = 1.0 - b
                wgtr[0, sl] = ia * ib
                wgtr[1, sl] = a * ib
                wgtr[2, sl] = ia * b
                wgtr[3, sl] = a * b

        def issue_gathers(idxr, rr):
            for j in range(4):
                pltpu.async_copy(table.at[idxr.at[j]], rr.at[j], sem_g)

        def drain_gathers(idxr, rr):
            for j in range(4):
                pltpu.make_async_copy(table.at[idxr.at[j]], rr.at[j], sem_g).wait()

        def blend(rr, wgtr, outr):
            hi_mask = jnp.full((L,), -65536, jnp.int32)
            UNROLL = 4
            for g in range(GPC):
                sl16 = pl.ds(g * L, L)
                pix = lanes + g * L
                wv0 = wgtr[0, sl16]
                wv1 = wgtr[1, sl16]
                wv2 = wgtr[2, sl16]
                wv3 = wgtr[3, sl16]
                jidx = [jnp.full((L,), j, jnp.int32) for j in range(4)]

                @plsc.parallel_loop(0, CH, unroll=UNROLL)
                def _(w):
                    ws = lax.broadcast(w, (L,))
                    v = [plsc.load_gather(rr, [jidx[j], pix, ws])
                         for j in range(4)]
                    lo = [plsc.bitcast(vv << 16, jnp.float32) for vv in v]
                    hi = [plsc.bitcast(vv & hi_mask, jnp.float32)
                          for vv in v]
                    lov = (wv0 * lo[0] + wv1 * lo[1]
                           + wv2 * lo[2] + wv3 * lo[3])
                    hiv = (wv0 * hi[0] + wv1 * hi[1]
                           + wv2 * hi[2] + wv3 * hi[3])
                    outr[2 * w, sl16] = lov
                    outr[2 * w + 1, sl16] = hiv

        def out_slice(ci):
            p0 = base + ci * K
            bb = p0 >> LOG_HW
            yx0 = pl.multiple_of(p0 - (bb << LOG_HW), K)
            return out_hbm.at[bb, :, pl.ds(yx0, K)]

        phase1(0, idxA, wgtA)
        issue_gathers(idxA, rA)
        half = NCHUNK // 2

        def step(cp, carry):
            c0 = 2 * cp
            phase1(c0 + 1, idxB, wgtB)
            issue_gathers(idxB, rB)
            drain_gathers(idxA, rA)

            @pl.when(cp >= 1)
            def _():
                pltpu.make_async_copy(outA, out_slice(c0 - 2), sem_w).wait()

            blend(rA, wgtA, outA)
            pltpu.async_copy(outA, out_slice(c0), sem_w)

            @pl.when(cp < half - 1)
            def _():
                phase1(c0 + 2, idxA, wgtA)
                issue_gathers(idxA, rA)

            drain_gathers(idxB, rB)

            @pl.when(cp >= 1)
            def _():
                pltpu.make_async_copy(outB, out_slice(c0 - 1), sem_w).wait()

            blend(rB, wgtB, outB)
            pltpu.async_copy(outB, out_slice(c0 + 1), sem_w)
            return carry

        lax.fori_loop(0, half, step, 0)
        pltpu.make_async_copy(outA, out_slice(NCHUNK - 2), sem_w).wait()
        pltpu.make_async_copy(outB, out_slice(NCHUNK - 1), sem_w).wait()

    return warp


def kernel(input1, input2):
    B, C, H, W = input1.shape
    def rne16(v):
        u = lax.bitcast_convert_type(v, jnp.int32)
        r = u + jnp.int32(0x7FFF) + (lax.shift_right_logical(u, 16) & 1)
        return lax.shift_right_logical(r, 16)

    words = rne16(input1[:, 0::2]) | (rne16(input1[:, 1::2]) << 16)
    table = words.transpose(0, 2, 3, 1).reshape(B * H * W, C // 2)
    flow = input2.reshape(B * 2, H * W)
    out = _build_warp(B, C, H, W)(table, flow)
    return out.reshape(B, C, H, W)
